# baseline (speedup 1.0000x reference)
import functools

import jax
import jax.numpy as jnp
from jax import lax
from jax.experimental import pallas as pl
from jax.experimental.pallas import tpu as pltpu
from jax.experimental.pallas import tpu_sc as plsc

NC = 2
NS = 16
NW = NC * NS
LANES = 16
CHUNK = 128


def kernel(heads, rels, tails, ent_h_embs, ent_t_embs, rel_embs, rel_inv_embs):
    B = heads.shape[0]
    D = ent_h_embs.shape[1]
    W = 2 * D
    BPW = B // NW
    nchunk = BPW // CHUNK
    ngroup = CHUNK // LANES

    mesh = plsc.VectorSubcoreMesh(core_axis_name="c", subcore_axis_name="s")

    @functools.partial(
        pl.kernel,
        out_type=jax.ShapeDtypeStruct((B,), jnp.float32),
        mesh=mesh,
        compiler_params=pltpu.CompilerParams(
            use_tc_tiling_on_sc=False, needs_layout_passes=False),
        scratch_types=(
            [pltpu.VMEM((BPW,), jnp.int32)] * 3
            + [pltpu.VMEM((CHUNK, W), jnp.float32)] * 6
            + [pltpu.VMEM((BPW,), jnp.float32)]
            + [pltpu.SemaphoreType.DMA] * 2
        ),
    )
    def sc_kernel(heads_h, rels_h, tails_h, ent_h, rel_h, out_h,
                  h_idx, t_idx, r_idx,
                  hb0, tb0, rb0, hb1, tb1, rb1,
                  out_v, sem0, sem1):
        wid = lax.axis_index("s") * NC + lax.axis_index("c")
        base = wid * BPW
        pltpu.sync_copy(heads_h.at[pl.ds(base, BPW)], h_idx)
        pltpu.sync_copy(tails_h.at[pl.ds(base, BPW)], t_idx)
        pltpu.sync_copy(rels_h.at[pl.ds(base, BPW)], r_idx)

        bufs = [(hb0, tb0, rb0), (hb1, tb1, rb1)]
        sems = [sem0, sem1]

        def fire(ci):
            s = pl.ds(ci * CHUNK, CHUNK)
            hb, tb, rb = bufs[ci % 2]
            sem = sems[ci % 2]
            return [
                pltpu.async_copy(ent_h.at[h_idx.at[s]], hb, sem),
                pltpu.async_copy(ent_h.at[t_idx.at[s]], tb, sem),
                pltpu.async_copy(rel_h.at[r_idx.at[s]], rb, sem),
            ]

        lane = lax.iota(jnp.int32, LANES)

        def compute(ci):
            hb, tb, rb = bufs[ci % 2]

            def group_body(g, _):
                row = g * LANES + lane

                def dim_body(d, acc):
                    col = jnp.full((LANES,), d, jnp.int32)
                    col2 = col + D
                    hh = plsc.load_gather(hb, [row, col])
                    th = plsc.load_gather(hb, [row, col2])
                    ht = plsc.load_gather(tb, [row, col])
                    tt = plsc.load_gather(tb, [row, col2])
                    rr = plsc.load_gather(rb, [row, col])
                    ri = plsc.load_gather(rb, [row, col2])
                    return acc + hh * rr * tt + ht * ri * th

                acc = lax.fori_loop(0, D, dim_body,
                                    jnp.zeros((LANES,), jnp.float32))
                v = acc * jnp.float32(0.5)
                v = jnp.minimum(jnp.maximum(v, jnp.float32(-20.0)),
                                jnp.float32(20.0))
                plsc.store_scatter(out_v, [ci * CHUNK + g * LANES + lane], v)
                return _

            lax.fori_loop(0, ngroup, group_body, None)

        cps = fire(0)
        for ci in range(nchunk):
            nxt = fire(ci + 1) if ci + 1 < nchunk else None
            for cp in cps:
                cp.wait()
            compute(ci)
            cps = nxt

        pltpu.sync_copy(out_v, out_h.at[pl.ds(base, BPW)])

    heads = heads.astype(jnp.int32)
    rels = rels.astype(jnp.int32)
    tails = tails.astype(jnp.int32)
    ent_cat = _transpose_concat(ent_h_embs.T, ent_t_embs.T)
    rel_cat = _transpose_concat(rel_embs.T, rel_inv_embs.T)
    return sc_kernel(heads, rels, tails, ent_cat, rel_cat)


_TBLK = 512


def _transpose_concat(a_t, b_t):
    D, N = a_t.shape
    W = 2 * D
    grid = (N + _TBLK - 1) // _TBLK

    def tk(a_ref, b_ref, o_ref):
        ident = jnp.eye(D, dtype=jnp.float32)
        dn = (((0,), (0,)), ((), ()))
        o_ref[:, 0:D] = lax.dot_general(
            a_ref[...], ident, dn, preferred_element_type=jnp.float32)
        o_ref[:, D:W] = lax.dot_general(
            b_ref[...], ident, dn, preferred_element_type=jnp.float32)

    return pl.pallas_call(
        tk,
        grid=(grid,),
        in_specs=[
            pl.BlockSpec((D, _TBLK), lambda i: (0, i)),
            pl.BlockSpec((D, _TBLK), lambda i: (0, i)),
        ],
        out_specs=pl.BlockSpec((_TBLK, W), lambda i: (i, 0)),
        out_shape=jax.ShapeDtypeStruct((N, W), jnp.float32),
    )(a_t, b_t)


# speedup vs baseline: 2.4561x; 2.4561x over previous
import functools

import jax
import jax.numpy as jnp
from jax import lax
from jax.experimental import pallas as pl
from jax.experimental.pallas import tpu as pltpu
from jax.experimental.pallas import tpu_sc as plsc

NC = 2
NS = 16
NW = NC * NS
LANES = 16
CHUNK = 128


def kernel(heads, rels, tails, ent_h_embs, ent_t_embs, rel_embs, rel_inv_embs):
    B = heads.shape[0]
    D = ent_h_embs.shape[1]
    W = 2 * D
    BPW = B // NW
    nchunk = BPW // CHUNK
    ngroup = CHUNK // LANES

    mesh = plsc.VectorSubcoreMesh(core_axis_name="c", subcore_axis_name="s")

    @functools.partial(
        pl.kernel,
        out_type=jax.ShapeDtypeStruct((B,), jnp.float32),
        mesh=mesh,
        compiler_params=pltpu.CompilerParams(
            use_tc_tiling_on_sc=False, needs_layout_passes=False),
        scratch_types=(
            [pltpu.VMEM((BPW,), jnp.int32)] * 3
            + [pltpu.VMEM((CHUNK, W), jnp.float32)] * 6
            + [pltpu.VMEM((BPW,), jnp.float32)]
            + [pltpu.SemaphoreType.DMA] * 2
        ),
    )
    def sc_kernel(heads_h, rels_h, tails_h, ent_h, rel_h, out_h,
                  h_idx, t_idx, r_idx,
                  hb0, tb0, rb0, hb1, tb1, rb1,
                  out_v, sem0, sem1):
        wid = lax.axis_index("s") * NC + lax.axis_index("c")
        base = wid * BPW
        pltpu.sync_copy(heads_h.at[pl.ds(base, BPW)], h_idx)
        pltpu.sync_copy(tails_h.at[pl.ds(base, BPW)], t_idx)
        pltpu.sync_copy(rels_h.at[pl.ds(base, BPW)], r_idx)

        bufs = [(hb0, tb0, rb0), (hb1, tb1, rb1)]
        sems = [sem0, sem1]

        def fire(ci):
            s = pl.ds(ci * CHUNK, CHUNK)
            hb, tb, rb = bufs[ci % 2]
            sem = sems[ci % 2]
            return [
                pltpu.async_copy(ent_h.at[h_idx.at[s]], hb, sem),
                pltpu.async_copy(ent_h.at[t_idx.at[s]], tb, sem),
                pltpu.async_copy(rel_h.at[r_idx.at[s]], rb, sem),
            ]

        lane = lax.iota(jnp.int32, LANES)

        def compute(ci):
            hb, tb, rb = bufs[ci % 2]

            def group_body(g, _):
                row = g * LANES + lane

                def dim_body(d, acc):
                    col = jnp.full((LANES,), d, jnp.int32)
                    col2 = col + D
                    hh = plsc.load_gather(hb, [row, col])
                    th = plsc.load_gather(hb, [row, col2])
                    ht = plsc.load_gather(tb, [row, col])
                    tt = plsc.load_gather(tb, [row, col2])
                    rr = plsc.load_gather(rb, [row, col])
                    ri = plsc.load_gather(rb, [row, col2])
                    return acc + hh * rr * tt + ht * ri * th

                acc = lax.fori_loop(0, D, dim_body,
                                    jnp.zeros((LANES,), jnp.float32))
                v = acc * jnp.float32(0.5)
                v = jnp.minimum(jnp.maximum(v, jnp.float32(-20.0)),
                                jnp.float32(20.0))
                plsc.store_scatter(out_v, [ci * CHUNK + g * LANES + lane], v)
                return _

            lax.fori_loop(0, ngroup, group_body, None)

        cps = fire(0)
        for ci in range(nchunk):
            nxt = fire(ci + 1) if ci + 1 < nchunk else None
            for cp in cps:
                cp.wait()
            compute(ci)
            cps = nxt

        pltpu.sync_copy(out_v, out_h.at[pl.ds(base, BPW)])

    heads = heads.astype(jnp.int32)
    rels = rels.astype(jnp.int32)
    tails = tails.astype(jnp.int32)
    ent_cat = _transpose_concat(ent_h_embs.T, ent_t_embs.T)
    rel_cat = _transpose_concat(rel_embs.T, rel_inv_embs.T)
    return sc_kernel(heads, rels, tails, ent_cat, rel_cat)


_TBLK = 4096


def _transpose_concat(a_t, b_t):
    D, N = a_t.shape
    W = 2 * D
    grid = (N + _TBLK - 1) // _TBLK

    def tk(a_ref, b_ref, o_ref):
        o_ref[:, 0:D] = a_ref[...].T
        o_ref[:, D:W] = b_ref[...].T

    return pl.pallas_call(
        tk,
        grid=(grid,),
        in_specs=[
            pl.BlockSpec((D, _TBLK), lambda i: (0, i)),
            pl.BlockSpec((D, _TBLK), lambda i: (0, i)),
        ],
        out_specs=pl.BlockSpec((_TBLK, W), lambda i: (i, 0)),
        out_shape=jax.ShapeDtypeStruct((N, W), jnp.float32),
    )(a_t, b_t)


# speedup vs baseline: 2.8737x; 1.1700x over previous
import functools

import jax
import jax.numpy as jnp
from jax import lax
from jax.experimental import pallas as pl
from jax.experimental.pallas import tpu as pltpu
from jax.experimental.pallas import tpu_sc as plsc

NC = 2
NS = 16
NW = NC * NS
LANES = 16
CHUNK = 128


def kernel(heads, rels, tails, ent_h_embs, ent_t_embs, rel_embs, rel_inv_embs):
    B = heads.shape[0]
    D = ent_h_embs.shape[1]
    W = 2 * D
    BPW = B // NW
    nchunk = BPW // CHUNK
    ngroup = CHUNK // LANES

    mesh = plsc.VectorSubcoreMesh(core_axis_name="c", subcore_axis_name="s")

    @functools.partial(
        pl.kernel,
        out_type=jax.ShapeDtypeStruct((B,), jnp.float32),
        mesh=mesh,
        compiler_params=pltpu.CompilerParams(
            use_tc_tiling_on_sc=False, needs_layout_passes=False),
        scratch_types=(
            [pltpu.VMEM((BPW,), jnp.int32)] * 3
            + [pltpu.VMEM((CHUNK, W), jnp.float32)] * 6
            + [pltpu.VMEM((BPW,), jnp.float32)]
            + [pltpu.SemaphoreType.DMA] * 2
        ),
    )
    def sc_kernel(heads_h, rels_h, tails_h, ent_h, rel_h, out_h,
                  h_idx, t_idx, r_idx,
                  hb0, tb0, rb0, hb1, tb1, rb1,
                  out_v, sem0, sem1):
        wid = lax.axis_index("s") * NC + lax.axis_index("c")
        base = wid * BPW
        pltpu.sync_copy(heads_h.at[pl.ds(base, BPW)], h_idx)
        pltpu.sync_copy(tails_h.at[pl.ds(base, BPW)], t_idx)
        pltpu.sync_copy(rels_h.at[pl.ds(base, BPW)], r_idx)

        bufs = [(hb0, tb0, rb0), (hb1, tb1, rb1)]
        sems = [sem0, sem1]

        def fire(ci):
            s = pl.ds(ci * CHUNK, CHUNK)
            hb, tb, rb = bufs[ci % 2]
            sem = sems[ci % 2]
            return [
                pltpu.async_copy(ent_h.at[h_idx.at[s]], hb, sem),
                pltpu.async_copy(ent_h.at[t_idx.at[s]], tb, sem),
                pltpu.async_copy(rel_h.at[r_idx.at[s]], rb, sem),
            ]

        lane = lax.iota(jnp.int32, LANES)

        def compute(ci):
            hb, tb, rb = bufs[ci % 2]

            def group_body(g, _):
                row = g * LANES + lane

                def dim_body(d, acc):
                    col = jnp.full((LANES,), d, jnp.int32)
                    col2 = col + D
                    hh = plsc.load_gather(hb, [row, col])
                    th = plsc.load_gather(hb, [row, col2])
                    ht = plsc.load_gather(tb, [row, col])
                    tt = plsc.load_gather(tb, [row, col2])
                    rr = plsc.load_gather(rb, [row, col])
                    ri = plsc.load_gather(rb, [row, col2])
                    return acc + hh * rr * tt + ht * ri * th

                acc = lax.fori_loop(0, D, dim_body,
                                    jnp.zeros((LANES,), jnp.float32))
                v = acc * jnp.float32(0.5)
                v = jnp.minimum(jnp.maximum(v, jnp.float32(-20.0)),
                                jnp.float32(20.0))
                plsc.store_scatter(out_v, [ci * CHUNK + g * LANES + lane], v)
                return _

            lax.fori_loop(0, ngroup, group_body, None)

        cps = fire(0)
        for ci in range(nchunk):
            nxt = fire(ci + 1) if ci + 1 < nchunk else None
            for cp in cps:
                cp.wait()
            compute(ci)
            cps = nxt

        pltpu.sync_copy(out_v, out_h.at[pl.ds(base, BPW)])

    heads = heads.astype(jnp.int32)
    rels = rels.astype(jnp.int32)
    tails = tails.astype(jnp.int32)
    ent_cat = _transpose_concat(ent_h_embs.T, ent_t_embs.T)
    rel_cat = _transpose_concat(rel_embs.T, rel_inv_embs.T)
    return sc_kernel(heads, rels, tails, ent_cat, rel_cat)


_TBLK = 16384


def _transpose_concat(a_t, b_t):
    D, N = a_t.shape
    W = 2 * D
    grid = (N + _TBLK - 1) // _TBLK

    def tk(a_ref, b_ref, o_ref):
        o_ref[:, 0:D] = a_ref[...].T
        o_ref[:, D:W] = b_ref[...].T

    return pl.pallas_call(
        tk,
        grid=(grid,),
        in_specs=[
            pl.BlockSpec((D, _TBLK), lambda i: (0, i)),
            pl.BlockSpec((D, _TBLK), lambda i: (0, i)),
        ],
        out_specs=pl.BlockSpec((_TBLK, W), lambda i: (i, 0)),
        out_shape=jax.ShapeDtypeStruct((N, W), jnp.float32),
    )(a_t, b_t)


# speedup vs baseline: 2.9534x; 1.0277x over previous
import functools

import jax
import jax.numpy as jnp
from jax import lax
from jax.experimental import pallas as pl
from jax.experimental.pallas import tpu as pltpu
from jax.experimental.pallas import tpu_sc as plsc

NC = 2
NS = 16
NW = NC * NS
LANES = 16
CHUNK = 128


def kernel(heads, rels, tails, ent_h_embs, ent_t_embs, rel_embs, rel_inv_embs):
    B = heads.shape[0]
    D = ent_h_embs.shape[1]
    W = 2 * D
    BPW = B // NW
    nchunk = BPW // CHUNK
    ngroup = CHUNK // LANES

    mesh = plsc.VectorSubcoreMesh(core_axis_name="c", subcore_axis_name="s")

    @functools.partial(
        pl.kernel,
        out_type=jax.ShapeDtypeStruct((B,), jnp.float32),
        mesh=mesh,
        compiler_params=pltpu.CompilerParams(
            use_tc_tiling_on_sc=False, needs_layout_passes=False),
        scratch_types=(
            [pltpu.VMEM((BPW,), jnp.int32)] * 3
            + [pltpu.VMEM((CHUNK, W), jnp.float32)] * 6
            + [pltpu.VMEM((BPW,), jnp.float32)]
            + [pltpu.SemaphoreType.DMA] * 2
        ),
    )
    def sc_kernel(heads_h, rels_h, tails_h, ent_h, rel_h, out_h,
                  h_idx, t_idx, r_idx,
                  hb0, tb0, rb0, hb1, tb1, rb1,
                  out_v, sem0, sem1):
        wid = lax.axis_index("s") * NC + lax.axis_index("c")
        base = wid * BPW
        pltpu.sync_copy(heads_h.at[pl.ds(base, BPW)], h_idx)
        pltpu.sync_copy(tails_h.at[pl.ds(base, BPW)], t_idx)
        pltpu.sync_copy(rels_h.at[pl.ds(base, BPW)], r_idx)

        bufs = [(hb0, tb0, rb0), (hb1, tb1, rb1)]
        sems = [sem0, sem1]

        def fire(ci):
            s = pl.ds(ci * CHUNK, CHUNK)
            hb, tb, rb = bufs[ci % 2]
            sem = sems[ci % 2]
            return [
                pltpu.async_copy(ent_h.at[h_idx.at[s]], hb, sem),
                pltpu.async_copy(ent_h.at[t_idx.at[s]], tb, sem),
                pltpu.async_copy(rel_h.at[r_idx.at[s]], rb, sem),
            ]

        lane = lax.iota(jnp.int32, LANES)

        def compute(ci):
            hb, tb, rb = bufs[ci % 2]

            def group_body(g, _):
                row = g * LANES + lane

                def dim_body(d, acc):
                    col = jnp.full((LANES,), d, jnp.int32)
                    col2 = col + D
                    hh = plsc.load_gather(hb, [row, col])
                    th = plsc.load_gather(hb, [row, col2])
                    ht = plsc.load_gather(tb, [row, col])
                    tt = plsc.load_gather(tb, [row, col2])
                    rr = plsc.load_gather(rb, [row, col])
                    ri = plsc.load_gather(rb, [row, col2])
                    return acc + hh * rr * tt + ht * ri * th

                acc = lax.fori_loop(0, D, dim_body,
                                    jnp.zeros((LANES,), jnp.float32),
                                    unroll=8)
                v = acc * jnp.float32(0.5)
                v = jnp.minimum(jnp.maximum(v, jnp.float32(-20.0)),
                                jnp.float32(20.0))
                plsc.store_scatter(out_v, [ci * CHUNK + g * LANES + lane], v)
                return _

            lax.fori_loop(0, ngroup, group_body, None)

        cps = fire(0)
        for ci in range(nchunk):
            nxt = fire(ci + 1) if ci + 1 < nchunk else None
            for cp in cps:
                cp.wait()
            compute(ci)
            cps = nxt

        pltpu.sync_copy(out_v, out_h.at[pl.ds(base, BPW)])

    heads = heads.astype(jnp.int32)
    rels = rels.astype(jnp.int32)
    tails = tails.astype(jnp.int32)
    ent_cat = _transpose_concat(ent_h_embs.T, ent_t_embs.T)
    rel_cat = _transpose_concat(rel_embs.T, rel_inv_embs.T)
    return sc_kernel(heads, rels, tails, ent_cat, rel_cat)


_TBLK = 16384


def _transpose_concat(a_t, b_t):
    D, N = a_t.shape
    W = 2 * D
    grid = (N + _TBLK - 1) // _TBLK

    def tk(a_ref, b_ref, o_ref):
        o_ref[:, 0:D] = a_ref[...].T
        o_ref[:, D:W] = b_ref[...].T

    return pl.pallas_call(
        tk,
        grid=(grid,),
        in_specs=[
            pl.BlockSpec((D, _TBLK), lambda i: (0, i)),
            pl.BlockSpec((D, _TBLK), lambda i: (0, i)),
        ],
        out_specs=pl.BlockSpec((_TBLK, W), lambda i: (i, 0)),
        out_shape=jax.ShapeDtypeStruct((N, W), jnp.float32),
    )(a_t, b_t)


# speedup vs baseline: 2.9583x; 1.0017x over previous
import functools

import jax
import jax.numpy as jnp
from jax import lax
from jax.experimental import pallas as pl
from jax.experimental.pallas import tpu as pltpu
from jax.experimental.pallas import tpu_sc as plsc

NC = 2
NS = 16
NW = NC * NS
LANES = 16
CHUNK = 128


def kernel(heads, rels, tails, ent_h_embs, ent_t_embs, rel_embs, rel_inv_embs):
    B = heads.shape[0]
    D = ent_h_embs.shape[1]
    W = 2 * D
    BPW = B // NW
    nchunk = BPW // CHUNK
    ngroup = CHUNK // LANES

    mesh = plsc.VectorSubcoreMesh(core_axis_name="c", subcore_axis_name="s")

    @functools.partial(
        pl.kernel,
        out_type=jax.ShapeDtypeStruct((B,), jnp.float32),
        mesh=mesh,
        compiler_params=pltpu.CompilerParams(
            use_tc_tiling_on_sc=False, needs_layout_passes=False),
        scratch_types=(
            [pltpu.VMEM((BPW,), jnp.int32)] * 3
            + [pltpu.VMEM((CHUNK, W), jnp.float32)] * 6
            + [pltpu.VMEM((BPW,), jnp.float32)]
            + [pltpu.SemaphoreType.DMA] * 2
        ),
    )
    def sc_kernel(heads_h, rels_h, tails_h, ent_h, rel_h, out_h,
                  h_idx, t_idx, r_idx,
                  hb0, tb0, rb0, hb1, tb1, rb1,
                  out_v, sem0, sem1):
        wid = lax.axis_index("s") * NC + lax.axis_index("c")
        base = wid * BPW
        pltpu.sync_copy(heads_h.at[pl.ds(base, BPW)], h_idx)
        pltpu.sync_copy(tails_h.at[pl.ds(base, BPW)], t_idx)
        pltpu.sync_copy(rels_h.at[pl.ds(base, BPW)], r_idx)

        bufs = [(hb0, tb0, rb0), (hb1, tb1, rb1)]
        sems = [sem0, sem1]

        def fire(ci):
            s = pl.ds(ci * CHUNK, CHUNK)
            hb, tb, rb = bufs[ci % 2]
            sem = sems[ci % 2]
            return [
                pltpu.async_copy(ent_h.at[h_idx.at[s]], hb, sem),
                pltpu.async_copy(ent_h.at[t_idx.at[s]], tb, sem),
                pltpu.async_copy(rel_h.at[r_idx.at[s]], rb, sem),
            ]

        lane = lax.iota(jnp.int32, LANES)

        def compute(ci):
            hb, tb, rb = bufs[ci % 2]

            def group_body(g, _):
                row = g * LANES + lane

                def dim_body(d, acc):
                    col = jnp.full((LANES,), d, jnp.int32)
                    col2 = col + D
                    hh = plsc.load_gather(hb, [row, col])
                    th = plsc.load_gather(hb, [row, col2])
                    ht = plsc.load_gather(tb, [row, col])
                    tt = plsc.load_gather(tb, [row, col2])
                    rr = plsc.load_gather(rb, [row, col])
                    ri = plsc.load_gather(rb, [row, col2])
                    return acc + hh * rr * tt + ht * ri * th

                acc = lax.fori_loop(0, D, dim_body,
                                    jnp.zeros((LANES,), jnp.float32),
                                    unroll=8)
                v = acc * jnp.float32(0.5)
                v = jnp.minimum(jnp.maximum(v, jnp.float32(-20.0)),
                                jnp.float32(20.0))
                plsc.store_scatter(out_v, [ci * CHUNK + g * LANES + lane], v)
                return _

            lax.fori_loop(0, ngroup, group_body, None)

        cps = fire(0)
        for ci in range(nchunk):
            nxt = fire(ci + 1) if ci + 1 < nchunk else None
            for cp in cps:
                cp.wait()
            compute(ci)
            cps = nxt

        pltpu.sync_copy(out_v, out_h.at[pl.ds(base, BPW)])

    heads = heads.astype(jnp.int32)
    rels = rels.astype(jnp.int32)
    tails = tails.astype(jnp.int32)
    ent_cat = _transpose_concat(ent_h_embs.T, ent_t_embs.T)
    rel_cat = _transpose_concat(rel_embs.T, rel_inv_embs.T)
    return sc_kernel(heads, rels, tails, ent_cat, rel_cat)


_TBLK = 20480


def _transpose_concat(a_t, b_t):
    D, N = a_t.shape
    W = 2 * D
    grid = (N + _TBLK - 1) // _TBLK

    def tk(a_ref, b_ref, o_ref):
        o_ref[:, 0:D] = a_ref[...].T
        o_ref[:, D:W] = b_ref[...].T

    return pl.pallas_call(
        tk,
        grid=(grid,),
        in_specs=[
            pl.BlockSpec((D, _TBLK), lambda i: (0, i)),
            pl.BlockSpec((D, _TBLK), lambda i: (0, i)),
        ],
        out_specs=pl.BlockSpec((_TBLK, W), lambda i: (i, 0)),
        out_shape=jax.ShapeDtypeStruct((N, W), jnp.float32),
    )(a_t, b_t)


# speedup vs baseline: 2.9751x; 1.0057x over previous
import functools

import jax
import jax.numpy as jnp
from jax import lax
from jax.experimental import pallas as pl
from jax.experimental.pallas import tpu as pltpu
from jax.experimental.pallas import tpu_sc as plsc

NC = 2
NS = 16
NW = NC * NS
LANES = 16
CHUNK = 128


def kernel(heads, rels, tails, ent_h_embs, ent_t_embs, rel_embs, rel_inv_embs):
    B = heads.shape[0]
    D = ent_h_embs.shape[1]
    W = 2 * D
    BPW = B // NW
    nchunk = BPW // CHUNK
    ngroup = CHUNK // LANES

    mesh = plsc.VectorSubcoreMesh(core_axis_name="c", subcore_axis_name="s")

    @functools.partial(
        pl.kernel,
        out_type=jax.ShapeDtypeStruct((B,), jnp.float32),
        mesh=mesh,
        compiler_params=pltpu.CompilerParams(
            use_tc_tiling_on_sc=False, needs_layout_passes=False),
        scratch_types=(
            [pltpu.VMEM((BPW,), jnp.int32)] * 3
            + [pltpu.VMEM((CHUNK, W), jnp.float32)] * 6
            + [pltpu.VMEM((BPW,), jnp.float32)]
            + [pltpu.SemaphoreType.DMA] * 2
        ),
    )
    def sc_kernel(heads_h, rels_h, tails_h, ent_h, rel_h, out_h,
                  h_idx, t_idx, r_idx,
                  hb0, tb0, rb0, hb1, tb1, rb1,
                  out_v, sem0, sem1):
        wid = lax.axis_index("s") * NC + lax.axis_index("c")
        base = wid * BPW
        pltpu.sync_copy(heads_h.at[pl.ds(base, BPW)], h_idx)
        pltpu.sync_copy(tails_h.at[pl.ds(base, BPW)], t_idx)
        pltpu.sync_copy(rels_h.at[pl.ds(base, BPW)], r_idx)

        bufs = [(hb0, tb0, rb0), (hb1, tb1, rb1)]
        sems = [sem0, sem1]

        def fire(ci):
            s = pl.ds(ci * CHUNK, CHUNK)
            hb, tb, rb = bufs[ci % 2]
            sem = sems[ci % 2]
            return [
                pltpu.async_copy(ent_h.at[h_idx.at[s]], hb, sem),
                pltpu.async_copy(ent_h.at[t_idx.at[s]], tb, sem),
                pltpu.async_copy(rel_h.at[r_idx.at[s]], rb, sem),
            ]

        lane = lax.iota(jnp.int32, LANES)

        def compute(ci):
            hb, tb, rb = bufs[ci % 2]

            def group_body(g, _):
                row = g * LANES + lane

                def dim_body(d, acc):
                    col = jnp.full((LANES,), d, jnp.int32)
                    col2 = col + D
                    hh = plsc.load_gather(hb, [row, col])
                    th = plsc.load_gather(hb, [row, col2])
                    ht = plsc.load_gather(tb, [row, col])
                    tt = plsc.load_gather(tb, [row, col2])
                    rr = plsc.load_gather(rb, [row, col])
                    ri = plsc.load_gather(rb, [row, col2])
                    return acc + hh * rr * tt + ht * ri * th

                acc = lax.fori_loop(0, D, dim_body,
                                    jnp.zeros((LANES,), jnp.float32),
                                    unroll=16)
                v = acc * jnp.float32(0.5)
                v = jnp.minimum(jnp.maximum(v, jnp.float32(-20.0)),
                                jnp.float32(20.0))
                plsc.store_scatter(out_v, [ci * CHUNK + g * LANES + lane], v)
                return _

            lax.fori_loop(0, ngroup, group_body, None)

        cps = fire(0)
        for ci in range(nchunk):
            nxt = fire(ci + 1) if ci + 1 < nchunk else None
            for cp in cps:
                cp.wait()
            compute(ci)
            cps = nxt

        pltpu.sync_copy(out_v, out_h.at[pl.ds(base, BPW)])

    heads = heads.astype(jnp.int32)
    rels = rels.astype(jnp.int32)
    tails = tails.astype(jnp.int32)
    ent_cat = _transpose_concat(ent_h_embs.T, ent_t_embs.T)
    rel_cat = _transpose_concat(rel_embs.T, rel_inv_embs.T)
    return sc_kernel(heads, rels, tails, ent_cat, rel_cat)


_TBLK = 20480


def _transpose_concat(a_t, b_t):
    D, N = a_t.shape
    W = 2 * D
    grid = (N + _TBLK - 1) // _TBLK

    def tk(a_ref, b_ref, o_ref):
        o_ref[:, 0:D] = a_ref[...].T
        o_ref[:, D:W] = b_ref[...].T

    return pl.pallas_call(
        tk,
        grid=(grid,),
        in_specs=[
            pl.BlockSpec((D, _TBLK), lambda i: (0, i)),
            pl.BlockSpec((D, _TBLK), lambda i: (0, i)),
        ],
        out_specs=pl.BlockSpec((_TBLK, W), lambda i: (i, 0)),
        out_shape=jax.ShapeDtypeStruct((N, W), jnp.float32),
    )(a_t, b_t)
